# baseline (speedup 1.0000x reference)
import dataclasses
import functools

import jax
import jax.numpy as jnp
from jax import lax
from jax.experimental import pallas as pl
from jax.experimental.pallas import tpu as pltpu
from jax.experimental.pallas import tpu_sc as plsc

_NC = 2
_NS = 16
_LANES = 16


def _pick_chunk(per_sub: int) -> int:
    for k in range(128, 0, -8):
        if per_sub % k == 0:
            return k
    raise ValueError(f"no valid chunk size for {per_sub}")


def _make_sc_spmm(N, E, H):
    per_sub = E // _NS
    assert per_sub * _NS == E
    K = _pick_chunk(per_sub)
    n_chunks = per_sub // K
    rows_per_sub = (N // (8 * _NS)) * 8
    tail_rows = N - _NS * rows_per_sub
    assert tail_rows % 8 == 0 and tail_rows <= K
    n_zfull, z_rem = divmod(rows_per_sub, K)
    f32 = jnp.float32

    mesh = plsc.VectorSubcoreMesh(core_axis_name="c", subcore_axis_name="s")
    cp = pltpu.CompilerParams()
    if "needs_layout_passes" in pltpu.CompilerParams.__dataclass_fields__:
        cp = dataclasses.replace(cp, needs_layout_passes=False)

    @functools.partial(
        pl.kernel,
        compiler_params=cp,
        out_type=(
            jax.ShapeDtypeStruct((N, H), f32),
            jax.ShapeDtypeStruct((N, H), f32),
        ),
        mesh=mesh,
        scratch_types=(
            [pltpu.VMEM_SHARED((N, H), f32)]
            + [pltpu.VMEM((K,), jnp.int32)] * 3
            + [pltpu.VMEM((K,), jnp.int32)] * 3
            + [pltpu.VMEM((K,), f32)] * 3
            + [pltpu.VMEM((K, H), f32)] * 3
            + [pltpu.SemaphoreType.DMA] * 9
        ),
    )
    def sc_spmm(lo_hbm, hi_hbm, row_hbm, col_hbm, w_hbm,
                out_lo, out_hi, acc, *scr):
        c = lax.axis_index("c")
        s = lax.axis_index("s")
        my_rows = s * rows_per_sub
        zero16 = jnp.zeros((_LANES,), f32)
        rowv = scr[0:3]
        colv = scr[3:6]
        wv = scr[6:9]
        rows = scr[9:12]
        sem_i = scr[12:15]
        sem_g = scr[15:18]
        sem_s = scr[18:21]
        rows0 = rows[0]

        def run(tbl_hbm, out_hbm):
            base0 = s * per_sub

            def _idx_descs(i, p, make):
                base = base0 + i * K
                return (
                    make(row_hbm.at[pl.ds(base, K)], rowv[p], sem_i[p]),
                    make(col_hbm.at[pl.ds(base, K)], colv[p], sem_i[p]),
                    make(w_hbm.at[pl.ds(base, K)], wv[p], sem_i[p]),
                )

            def idx_issue(i, p):
                _idx_descs(i, p, pltpu.async_copy)

            def idx_wait(i, p):
                for d in _idx_descs(i, p, pltpu.make_async_copy):
                    d.wait()

            def scale(p):
                @plsc.parallel_loop(0, K, unroll=8)
                def _(e):
                    e16 = jnp.full((_LANES,), e, jnp.int32)
                    wb = plsc.load_gather(wv[p], [e16])
                    for j in range(H // _LANES):
                        sl = pl.ds(j * _LANES, _LANES)
                        rows[p][e, sl] = rows[p][e, sl] * wb

            @pl.loop(0, K)
            def _(r):
                for j in range(H // _LANES):
                    rows0[r, pl.ds(j * _LANES, _LANES)] = zero16

            for t in range(n_zfull):
                pltpu.sync_copy(rows0, acc.at[pl.ds(my_rows + t * K, K)])
            if z_rem:
                pltpu.sync_copy(rows0.at[pl.ds(0, z_rem)],
                                acc.at[pl.ds(my_rows + n_zfull * K, z_rem)])
            if tail_rows:
                @pl.when(s == _NS - 1)
                def _():
                    pltpu.sync_copy(rows0.at[pl.ds(0, tail_rows)],
                                    acc.at[pl.ds(_NS * rows_per_sub, tail_rows)])
            plsc.subcore_barrier()

            def step(i, p):
                nx = (p + 1) % 3
                pv = (p + 2) % 3

                @pl.when(i >= 2)
                def _():
                    pltpu.make_async_copy(rows[nx], acc.at[rowv[nx]],
                                          sem_s[nx]).wait()

                @pl.when(i + 1 < n_chunks)
                def _():
                    idx_wait(i + 1, nx)
                    pltpu.async_copy(tbl_hbm.at[colv[nx]], rows[nx], sem_g[nx])

                pltpu.make_async_copy(tbl_hbm.at[colv[p]], rows[p],
                                      sem_g[p]).wait()
                scale(p)
                pltpu.async_copy(rows[p], acc.at[rowv[p]], sem_s[p], add=True)

                @pl.when(i + 2 < n_chunks)
                def _():
                    idx_issue(i + 2, pv)

            idx_issue(0, 0)
            idx_wait(0, 0)
            pltpu.async_copy(tbl_hbm.at[colv[0]], rows[0], sem_g[0])
            if n_chunks > 1:
                idx_issue(1, 1)

            n_tri = n_chunks - (n_chunks % 3)

            @pl.loop(0, n_tri, step=3)
            def _(g):
                step(g, 0)
                step(g + 1, 1)
                step(g + 2, 2)

            for i in range(n_tri, n_chunks):
                step(jnp.int32(i), i % 3)

            for i in (n_chunks - 2, n_chunks - 1):
                pltpu.make_async_copy(rows[i % 3], acc.at[rowv[i % 3]],
                                      sem_s[i % 3]).wait()
            plsc.subcore_barrier()
            pltpu.sync_copy(acc.at[pl.ds(my_rows, rows_per_sub)],
                            out_hbm.at[pl.ds(my_rows, rows_per_sub)])
            if tail_rows:
                @pl.when(s == _NS - 1)
                def _():
                    t0 = _NS * rows_per_sub
                    pltpu.sync_copy(acc.at[pl.ds(t0, tail_rows)],
                                    out_hbm.at[pl.ds(t0, tail_rows)])

        @pl.when(c == 0)
        def _():
            run(lo_hbm, out_lo)

        @pl.when(c == 1)
        def _():
            run(hi_hbm, out_hi)

    return sc_spmm


def _tc_head(agg_lo, agg_hi, Wl, Wh, N, H, D_OUT):
    bn = 1000 if N % 1000 == 0 else 8
    assert N % bn == 0

    def body(lo_ref, hi_ref, wl_ref, wh_ref, o_ref):
        h = jnp.dot(lo_ref[...], wl_ref[...], preferred_element_type=jnp.float32)
        h = h + jnp.dot(hi_ref[...], wh_ref[...], preferred_element_type=jnp.float32)
        h = jnp.tanh(h)
        norm = jnp.sqrt(jnp.sum(h * h, axis=1, keepdims=True))
        o_ref[...] = h / jnp.maximum(norm, 1e-12)

    return pl.pallas_call(
        body,
        grid=(N // bn,),
        in_specs=[
            pl.BlockSpec((bn, H), lambda i: (i, 0)),
            pl.BlockSpec((bn, H), lambda i: (i, 0)),
            pl.BlockSpec((H, D_OUT), lambda i: (0, 0)),
            pl.BlockSpec((H, D_OUT), lambda i: (0, 0)),
        ],
        out_specs=pl.BlockSpec((bn, D_OUT), lambda i: (i, 0)),
        out_shape=jax.ShapeDtypeStruct((N, D_OUT), jnp.float32),
    )(agg_lo, agg_hi, Wl, Wh)


def kernel(emb, edge_index, edge_weight, W):
    N, D_IN = emb.shape
    D_OUT = W.shape[0]
    E = edge_weight.shape[0]
    H = D_IN // 2

    row = edge_index[0]
    col = edge_index[1]
    emb_lo = emb[:, :H]
    emb_hi = emb[:, H:]

    sc_spmm = _make_sc_spmm(N, E, H)
    agg_lo, agg_hi = sc_spmm(emb_lo, emb_hi, row, col, edge_weight)

    Wl = W[:, :H].T
    Wh = W[:, H:].T
    return _tc_head(agg_lo, agg_hi, Wl, Wh, N, H, D_OUT)


# speedup vs baseline: 1.0471x; 1.0471x over previous
import dataclasses
import functools

import jax
import jax.numpy as jnp
from jax import lax
from jax.experimental import pallas as pl
from jax.experimental.pallas import tpu as pltpu
from jax.experimental.pallas import tpu_sc as plsc

_NC = 2
_NS = 16
_LANES = 16


def _pick_chunk(per_sub: int) -> int:
    for k in range(128, 0, -8):
        if per_sub % k == 0:
            return k
    raise ValueError(f"no valid chunk size for {per_sub}")


def _make_sc_spmm(N, E, H):
    per_sub = E // _NS
    assert per_sub * _NS == E
    K = _pick_chunk(per_sub)
    n_chunks = per_sub // K
    rows_per_sub = (N // (8 * _NS)) * 8
    tail_rows = N - _NS * rows_per_sub
    assert tail_rows % 8 == 0 and tail_rows <= K
    n_zfull, z_rem = divmod(rows_per_sub, K)
    f32 = jnp.float32

    mesh = plsc.VectorSubcoreMesh(core_axis_name="c", subcore_axis_name="s")
    cp = pltpu.CompilerParams()
    if "needs_layout_passes" in pltpu.CompilerParams.__dataclass_fields__:
        cp = dataclasses.replace(cp, needs_layout_passes=False)

    @functools.partial(
        pl.kernel,
        compiler_params=cp,
        out_type=(
            jax.ShapeDtypeStruct((N, H), f32),
            jax.ShapeDtypeStruct((N, H), f32),
        ),
        mesh=mesh,
        scratch_types=(
            [pltpu.VMEM_SHARED((N, H), f32)]
            + [pltpu.VMEM((K,), jnp.int32)] * 4
            + [pltpu.VMEM((K,), jnp.int32)] * 4
            + [pltpu.VMEM((K,), f32)] * 4
            + [pltpu.VMEM((K, H), f32)] * 4
            + [pltpu.SemaphoreType.DMA] * 12
        ),
    )
    def sc_spmm(lo_hbm, hi_hbm, row_hbm, col_hbm, w_hbm,
                out_lo, out_hi, acc, *scr):
        c = lax.axis_index("c")
        s = lax.axis_index("s")
        my_rows = s * rows_per_sub
        zero16 = jnp.zeros((_LANES,), f32)
        rowv = scr[0:4]
        colv = scr[4:8]
        wv = scr[8:12]
        rows = scr[12:16]
        sem_i = scr[16:20]
        sem_g = scr[20:24]
        sem_s = scr[24:28]
        rows0 = rows[0]

        def run(tbl_hbm, out_hbm):
            base0 = s * per_sub

            def _idx_descs(i, p, make):
                base = base0 + i * K
                return (
                    make(row_hbm.at[pl.ds(base, K)], rowv[p], sem_i[p]),
                    make(col_hbm.at[pl.ds(base, K)], colv[p], sem_i[p]),
                    make(w_hbm.at[pl.ds(base, K)], wv[p], sem_i[p]),
                )

            def idx_issue(i, p):
                _idx_descs(i, p, pltpu.async_copy)

            def idx_wait(i, p):
                for d in _idx_descs(i, p, pltpu.make_async_copy):
                    d.wait()

            def scale(p):
                @plsc.parallel_loop(0, K, unroll=4)
                def _(e):
                    e16 = jnp.full((_LANES,), e, jnp.int32)
                    wb = plsc.load_gather(wv[p], [e16])
                    for j in range(H // _LANES):
                        sl = pl.ds(j * _LANES, _LANES)
                        rows[p][e, sl] = rows[p][e, sl] * wb

            @pl.loop(0, K)
            def _(r):
                for j in range(H // _LANES):
                    rows0[r, pl.ds(j * _LANES, _LANES)] = zero16

            for t in range(n_zfull):
                pltpu.sync_copy(rows0, acc.at[pl.ds(my_rows + t * K, K)])
            if z_rem:
                pltpu.sync_copy(rows0.at[pl.ds(0, z_rem)],
                                acc.at[pl.ds(my_rows + n_zfull * K, z_rem)])
            if tail_rows:
                @pl.when(s == _NS - 1)
                def _():
                    pltpu.sync_copy(rows0.at[pl.ds(0, tail_rows)],
                                    acc.at[pl.ds(_NS * rows_per_sub, tail_rows)])
            plsc.subcore_barrier()

            def step(i, p):
                n2 = (p + 2) % 4
                n3 = (p + 3) % 4

                @pl.when(i >= 2)
                def _():
                    pltpu.make_async_copy(rows[n2], acc.at[rowv[n2]],
                                          sem_s[n2]).wait()

                @pl.when(i + 2 < n_chunks)
                def _():
                    idx_wait(i + 2, n2)
                    pltpu.async_copy(tbl_hbm.at[colv[n2]], rows[n2], sem_g[n2])

                pltpu.make_async_copy(tbl_hbm.at[colv[p]], rows[p],
                                      sem_g[p]).wait()
                scale(p)
                pltpu.async_copy(rows[p], acc.at[rowv[p]], sem_s[p], add=True)

                @pl.when(i + 3 < n_chunks)
                def _():
                    idx_issue(i + 3, n3)

            idx_issue(0, 0)
            idx_issue(1, 1)
            idx_issue(2, 2)
            idx_wait(0, 0)
            pltpu.async_copy(tbl_hbm.at[colv[0]], rows[0], sem_g[0])
            idx_wait(1, 1)
            pltpu.async_copy(tbl_hbm.at[colv[1]], rows[1], sem_g[1])

            n_quad = n_chunks - (n_chunks % 4)

            @pl.loop(0, n_quad, step=4)
            def _(g):
                step(g, 0)
                step(g + 1, 1)
                step(g + 2, 2)
                step(g + 3, 3)

            for i in range(n_quad, n_chunks):
                step(jnp.int32(i), i % 4)

            for i in (n_chunks - 2, n_chunks - 1):
                pltpu.make_async_copy(rows[i % 4], acc.at[rowv[i % 4]],
                                      sem_s[i % 4]).wait()
            plsc.subcore_barrier()
            pltpu.sync_copy(acc.at[pl.ds(my_rows, rows_per_sub)],
                            out_hbm.at[pl.ds(my_rows, rows_per_sub)])
            if tail_rows:
                @pl.when(s == _NS - 1)
                def _():
                    t0 = _NS * rows_per_sub
                    pltpu.sync_copy(acc.at[pl.ds(t0, tail_rows)],
                                    out_hbm.at[pl.ds(t0, tail_rows)])

        @pl.when(c == 0)
        def _():
            run(lo_hbm, out_lo)

        @pl.when(c == 1)
        def _():
            run(hi_hbm, out_hi)

    return sc_spmm


def _tc_head(agg_lo, agg_hi, Wl, Wh, N, H, D_OUT):
    bn = 1000 if N % 1000 == 0 else 8
    assert N % bn == 0

    def body(lo_ref, hi_ref, wl_ref, wh_ref, o_ref):
        h = jnp.dot(lo_ref[...], wl_ref[...], preferred_element_type=jnp.float32)
        h = h + jnp.dot(hi_ref[...], wh_ref[...], preferred_element_type=jnp.float32)
        h = jnp.tanh(h)
        norm = jnp.sqrt(jnp.sum(h * h, axis=1, keepdims=True))
        o_ref[...] = h / jnp.maximum(norm, 1e-12)

    return pl.pallas_call(
        body,
        grid=(N // bn,),
        in_specs=[
            pl.BlockSpec((bn, H), lambda i: (i, 0)),
            pl.BlockSpec((bn, H), lambda i: (i, 0)),
            pl.BlockSpec((H, D_OUT), lambda i: (0, 0)),
            pl.BlockSpec((H, D_OUT), lambda i: (0, 0)),
        ],
        out_specs=pl.BlockSpec((bn, D_OUT), lambda i: (i, 0)),
        out_shape=jax.ShapeDtypeStruct((N, D_OUT), jnp.float32),
    )(agg_lo, agg_hi, Wl, Wh)


def kernel(emb, edge_index, edge_weight, W):
    N, D_IN = emb.shape
    D_OUT = W.shape[0]
    E = edge_weight.shape[0]
    H = D_IN // 2

    row = edge_index[0]
    col = edge_index[1]
    emb_lo = emb[:, :H]
    emb_hi = emb[:, H:]

    sc_spmm = _make_sc_spmm(N, E, H)
    agg_lo, agg_hi = sc_spmm(emb_lo, emb_hi, row, col, edge_weight)

    Wl = W[:, :H].T
    Wh = W[:, H:].T
    return _tc_head(agg_lo, agg_hi, Wl, Wh, N, H, D_OUT)
